# baseline (speedup 1.0000x reference)
import functools

import jax
import jax.numpy as jnp
from jax import lax
from jax.experimental import pallas as pl
from jax.experimental.pallas import tpu as pltpu
from jax.experimental.pallas import tpu_sc as plsc

B, S, D_IN, D_LAT, K = 64, 1024, 256, 64, 1024
N = B * S

T = 1024
G = N // T
HALVES = 2
N_H = N // HALVES
G_H = G // HALVES

NUM_CORES = 2
NUM_SUBCORES = 16
NW = NUM_CORES * NUM_SUBCORES
PER_W = N_H // NW
CHUNK = 128
NCHUNK = PER_W // CHUNK
NPAIR = NCHUNK // 2


def _tc_idx_body(x_ref, w_enc_ref, b_enc_ref, cb_ref, idx_ref):
    x = x_ref[...]
    w = w_enc_ref[...]
    b = b_enc_ref[...]
    cb = cb_ref[...]
    z_e = jnp.dot(x, w, preferred_element_type=jnp.float32) + b
    scores_t = lax.dot_general(cb, z_e, (((1,), (1,)), ((), ())),
                               preferred_element_type=jnp.float32)
    z_sq = lax.dot_general(jnp.ones((1, D_LAT), jnp.float32), z_e * z_e,
                           (((1,), (1,)), ((), ())),
                           precision=lax.Precision.HIGHEST,
                           preferred_element_type=jnp.float32)
    cb_sq = jnp.sum(cb * cb, axis=1, keepdims=True)
    d = (z_sq - 2.0 * scores_t) + cb_sq
    m = jnp.min(d, axis=0, keepdims=True)
    mask = jnp.where(d == m, 1.0, 0.0)
    iota_row = lax.broadcasted_iota(jnp.int32, (1, K), 1).astype(jnp.float32)
    idx_f = lax.dot_general(iota_row, mask, (((1,), (0,)), ((), ())),
                            precision=lax.Precision.HIGHEST,
                            preferred_element_type=jnp.float32)
    idx = jnp.clip(idx_f.astype(jnp.int32), 0, K - 1)
    idx_ref[...] = idx[None]


def _tc_zq_body(x_ref, w_enc_ref, b_enc_ref, cb_ref, idx_ref,
                z_e_ref, z_q_ref):
    x = x_ref[...]
    w = w_enc_ref[...]
    b = b_enc_ref[...]
    cb = cb_ref[...]
    z_e_ref[...] = jnp.dot(x, w, preferred_element_type=jnp.float32) + b
    idx = idx_ref[0]
    iota = lax.broadcasted_iota(jnp.int32, (K, T), 0)
    onehot = jnp.where(iota == idx, 1.0, 0.0)
    z_q_ref[...] = lax.dot_general(onehot, cb, (((0,), (0,)), ((), ())),
                                   preferred_element_type=jnp.float32)


def _cdec_body(cb_ref, w_dec_ref, b_dec_ref, out_ref):
    out_ref[...] = (jnp.dot(cb_ref[...], w_dec_ref[...],
                            preferred_element_type=jnp.float32)
                    + b_dec_ref[...])


def _tc_idx_call(x2, W_enc, b_enc2, codebook, half):
    off = half * G_H
    return pl.pallas_call(
        _tc_idx_body,
        grid=(G_H,),
        in_specs=[
            pl.BlockSpec((T, D_IN), lambda i: (i + off, 0)),
            pl.BlockSpec((D_IN, D_LAT), lambda i: (0, 0)),
            pl.BlockSpec((1, D_LAT), lambda i: (0, 0)),
            pl.BlockSpec((K, D_LAT), lambda i: (0, 0)),
        ],
        out_specs=pl.BlockSpec((1, 1, T), lambda i: (i, 0, 0)),
        out_shape=jax.ShapeDtypeStruct((G_H, 1, T), jnp.int32),
    )(x2, W_enc, b_enc2, codebook)


def _sc_body(half_off, cdec_hbm, idx_hbm, xrec_hbm,
             idx_v, buf_a, buf_b,
             gsem_a, gsem_b, wsem_a, wsem_b):
    wid = lax.axis_index("s") * NUM_CORES + lax.axis_index("c")
    base = half_off + wid * PER_W
    pltpu.sync_copy(idx_hbm.at[pl.ds(wid * PER_W, PER_W)], idx_v)

    def idx_sl(j):
        return idx_v.at[pl.ds(j * CHUNK, CHUNK)]

    def fire_gather(j, buf, gsem):
        pltpu.async_copy(cdec_hbm.at[idx_sl(j)], buf, gsem)

    def drain_gather(j, buf, gsem):
        pltpu.make_async_copy(cdec_hbm.at[idx_sl(j)], buf, gsem).wait()

    def fire_write(j, buf, wsem):
        off = base + j * CHUNK
        pltpu.async_copy(buf, xrec_hbm.at[pl.ds(off, CHUNK)], wsem)

    def drain_write(j, buf, wsem):
        off = base + j * CHUNK
        pltpu.make_async_copy(buf, xrec_hbm.at[pl.ds(off, CHUNK)],
                              wsem).wait()

    fire_gather(0, buf_a, gsem_a)
    fire_gather(1, buf_b, gsem_b)

    def body(p, carry):
        j0 = 2 * p
        drain_gather(j0, buf_a, gsem_a)
        fire_write(j0, buf_a, wsem_a)
        drain_gather(j0 + 1, buf_b, gsem_b)
        fire_write(j0 + 1, buf_b, wsem_b)

        @pl.when(p < NPAIR - 1)
        def _():
            drain_write(j0, buf_a, wsem_a)
            fire_gather(j0 + 2, buf_a, gsem_a)
            drain_write(j0 + 1, buf_b, wsem_b)
            fire_gather(j0 + 3, buf_b, gsem_b)

        return carry

    lax.fori_loop(0, NPAIR, body, 0)
    drain_write(NCHUNK - 2, buf_a, wsem_a)
    drain_write(NCHUNK - 1, buf_b, wsem_b)


_SC_SCRATCH = [
    pltpu.VMEM((PER_W,), jnp.int32),
    pltpu.VMEM((CHUNK, D_IN), jnp.float32),
    pltpu.VMEM((CHUNK, D_IN), jnp.float32),
    pltpu.SemaphoreType.DMA,
    pltpu.SemaphoreType.DMA,
    pltpu.SemaphoreType.DMA,
    pltpu.SemaphoreType.DMA,
]


@functools.cache
def _make_sc_gather_first():
    mesh = plsc.VectorSubcoreMesh(core_axis_name="c", subcore_axis_name="s")

    @functools.partial(
        pl.kernel,
        mesh=mesh,
        out_type=jax.ShapeDtypeStruct((N_H, D_IN), jnp.float32),
        scratch_types=list(_SC_SCRATCH),
    )
    def _sc0(cdec_hbm, idx_hbm, xrec_hbm, *rest):
        _sc_body(0, cdec_hbm, idx_hbm, xrec_hbm, *rest)

    return _sc0


def kernel(x, W_enc, b_enc, W_dec, b_dec, codebook):
    x2 = x.reshape(N, D_IN)
    b_enc2 = b_enc.reshape(1, D_LAT)

    c_dec = pl.pallas_call(
        _cdec_body,
        out_shape=jax.ShapeDtypeStruct((K, D_IN), jnp.float32),
    )(codebook, W_dec, b_dec.reshape(1, D_IN))

    idx0 = _tc_idx_call(x2, W_enc, b_enc2, codebook, 0)
    xr0 = _make_sc_gather_first()(c_dec, idx0.reshape(N_H))
    idx1 = _tc_idx_call(x2, W_enc, b_enc2, codebook, 1)
    xr1 = _make_sc_gather_first()(c_dec, idx1.reshape(N_H))

    idx_all = jnp.concatenate([idx0, idx1], axis=0)

    z_e_flat, z_q_flat = pl.pallas_call(
        _tc_zq_body,
        grid=(G,),
        in_specs=[
            pl.BlockSpec((T, D_IN), lambda i: (i, 0)),
            pl.BlockSpec((D_IN, D_LAT), lambda i: (0, 0)),
            pl.BlockSpec((1, D_LAT), lambda i: (0, 0)),
            pl.BlockSpec((K, D_LAT), lambda i: (0, 0)),
            pl.BlockSpec((1, 1, T), lambda i: (i, 0, 0)),
        ],
        out_specs=[
            pl.BlockSpec((T, D_LAT), lambda i: (i, 0)),
            pl.BlockSpec((T, D_LAT), lambda i: (i, 0)),
        ],
        out_shape=[
            jax.ShapeDtypeStruct((N, D_LAT), jnp.float32),
            jax.ShapeDtypeStruct((N, D_LAT), jnp.float32),
        ],
    )(x2, W_enc, b_enc2, codebook, idx_all)

    x_rec_flat = jnp.concatenate([xr0, xr1], axis=0)

    return (z_e_flat.reshape(B, S, D_LAT),
            z_q_flat.reshape(B, S, D_LAT),
            x_rec_flat.reshape(B, S, D_IN),
            idx_all.reshape(B, S))


# speedup vs baseline: 1.2256x; 1.2256x over previous
import functools

import jax
import jax.numpy as jnp
from jax import lax
from jax.experimental import pallas as pl
from jax.experimental.pallas import tpu as pltpu
from jax.experimental.pallas import tpu_sc as plsc

B, S, D_IN, D_LAT, K = 64, 1024, 256, 64, 1024
N = B * S

T = 2048
G = N // T
HALVES = 2
N_H = N // HALVES
G_H = G // HALVES

NUM_CORES = 2
NUM_SUBCORES = 16
NW = NUM_CORES * NUM_SUBCORES
PER_W = N_H // NW
CHUNK = 128
NCHUNK = PER_W // CHUNK
NPAIR = NCHUNK // 2


def _tc_idx_body(x_ref, w_enc_ref, b_enc_ref, cb_ref, idx_ref):
    x = x_ref[...]
    w = w_enc_ref[...]
    b = b_enc_ref[...]
    cb = cb_ref[...]
    z_e = jnp.dot(x, w, preferred_element_type=jnp.float32) + b
    scores_t = lax.dot_general(cb, z_e, (((1,), (1,)), ((), ())),
                               preferred_element_type=jnp.float32)
    z_sq = lax.dot_general(jnp.ones((1, D_LAT), jnp.float32), z_e * z_e,
                           (((1,), (1,)), ((), ())),
                           precision=lax.Precision.HIGHEST,
                           preferred_element_type=jnp.float32)
    cb_sq = jnp.sum(cb * cb, axis=1, keepdims=True)
    d = (z_sq - 2.0 * scores_t) + cb_sq
    m = jnp.min(d, axis=0, keepdims=True)
    iota = lax.broadcasted_iota(jnp.int32, (K, T), 0)
    idx = jnp.min(jnp.where(d == m, iota, K), axis=0, keepdims=True)
    idx_ref[...] = idx[None]


def _tc_zq_body(x_ref, w_enc_ref, b_enc_ref, cb_ref, idx_ref,
                z_e_ref, z_q_ref):
    x = x_ref[...]
    w = w_enc_ref[...]
    b = b_enc_ref[...]
    cb = cb_ref[...]
    z_e_ref[...] = jnp.dot(x, w, preferred_element_type=jnp.float32) + b
    idx = idx_ref[0]
    iota = lax.broadcasted_iota(jnp.int32, (K, T), 0)
    onehot = jnp.where(iota == idx, 1.0, 0.0)
    z_q_ref[...] = lax.dot_general(onehot, cb, (((0,), (0,)), ((), ())),
                                   preferred_element_type=jnp.float32)


def _cdec_body(cb_ref, w_dec_ref, b_dec_ref, out_ref):
    out_ref[...] = (jnp.dot(cb_ref[...], w_dec_ref[...],
                            preferred_element_type=jnp.float32)
                    + b_dec_ref[...])


def _tc_idx_call(x2, W_enc, b_enc2, codebook, half):
    off = half * G_H
    return pl.pallas_call(
        _tc_idx_body,
        grid=(G_H,),
        in_specs=[
            pl.BlockSpec((T, D_IN), lambda i: (i + off, 0)),
            pl.BlockSpec((D_IN, D_LAT), lambda i: (0, 0)),
            pl.BlockSpec((1, D_LAT), lambda i: (0, 0)),
            pl.BlockSpec((K, D_LAT), lambda i: (0, 0)),
        ],
        out_specs=pl.BlockSpec((1, 1, T), lambda i: (i, 0, 0)),
        out_shape=jax.ShapeDtypeStruct((G_H, 1, T), jnp.int32),
    )(x2, W_enc, b_enc2, codebook)


def _sc_body(half_off, cdec_hbm, idx_hbm, xrec_hbm,
             idx_v, buf_a, buf_b,
             gsem_a, gsem_b, wsem_a, wsem_b):
    wid = lax.axis_index("s") * NUM_CORES + lax.axis_index("c")
    base = half_off + wid * PER_W
    pltpu.sync_copy(idx_hbm.at[pl.ds(wid * PER_W, PER_W)], idx_v)

    def idx_sl(j):
        return idx_v.at[pl.ds(j * CHUNK, CHUNK)]

    def fire_gather(j, buf, gsem):
        pltpu.async_copy(cdec_hbm.at[idx_sl(j)], buf, gsem)

    def drain_gather(j, buf, gsem):
        pltpu.make_async_copy(cdec_hbm.at[idx_sl(j)], buf, gsem).wait()

    def fire_write(j, buf, wsem):
        off = base + j * CHUNK
        pltpu.async_copy(buf, xrec_hbm.at[pl.ds(off, CHUNK)], wsem)

    def drain_write(j, buf, wsem):
        off = base + j * CHUNK
        pltpu.make_async_copy(buf, xrec_hbm.at[pl.ds(off, CHUNK)],
                              wsem).wait()

    fire_gather(0, buf_a, gsem_a)
    fire_gather(1, buf_b, gsem_b)

    def body(p, carry):
        j0 = 2 * p
        drain_gather(j0, buf_a, gsem_a)
        fire_write(j0, buf_a, wsem_a)
        drain_gather(j0 + 1, buf_b, gsem_b)
        fire_write(j0 + 1, buf_b, wsem_b)

        @pl.when(p < NPAIR - 1)
        def _():
            drain_write(j0, buf_a, wsem_a)
            fire_gather(j0 + 2, buf_a, gsem_a)
            drain_write(j0 + 1, buf_b, wsem_b)
            fire_gather(j0 + 3, buf_b, gsem_b)

        return carry

    lax.fori_loop(0, NPAIR, body, 0)
    drain_write(NCHUNK - 2, buf_a, wsem_a)
    drain_write(NCHUNK - 1, buf_b, wsem_b)


_SC_SCRATCH = [
    pltpu.VMEM((PER_W,), jnp.int32),
    pltpu.VMEM((CHUNK, D_IN), jnp.float32),
    pltpu.VMEM((CHUNK, D_IN), jnp.float32),
    pltpu.SemaphoreType.DMA,
    pltpu.SemaphoreType.DMA,
    pltpu.SemaphoreType.DMA,
    pltpu.SemaphoreType.DMA,
]


@functools.cache
def _make_sc_gather_first():
    mesh = plsc.VectorSubcoreMesh(core_axis_name="c", subcore_axis_name="s")

    @functools.partial(
        pl.kernel,
        mesh=mesh,
        out_type=jax.ShapeDtypeStruct((N_H, D_IN), jnp.float32),
        scratch_types=list(_SC_SCRATCH),
    )
    def _sc0(cdec_hbm, idx_hbm, xrec_hbm, *rest):
        _sc_body(0, cdec_hbm, idx_hbm, xrec_hbm, *rest)

    return _sc0


def kernel(x, W_enc, b_enc, W_dec, b_dec, codebook):
    x2 = x.reshape(N, D_IN)
    b_enc2 = b_enc.reshape(1, D_LAT)

    c_dec = pl.pallas_call(
        _cdec_body,
        out_shape=jax.ShapeDtypeStruct((K, D_IN), jnp.float32),
    )(codebook, W_dec, b_dec.reshape(1, D_IN))

    idx0 = _tc_idx_call(x2, W_enc, b_enc2, codebook, 0)
    xr0 = _make_sc_gather_first()(c_dec, idx0.reshape(N_H))
    idx1 = _tc_idx_call(x2, W_enc, b_enc2, codebook, 1)
    xr1 = _make_sc_gather_first()(c_dec, idx1.reshape(N_H))

    idx_all = jnp.concatenate([idx0, idx1], axis=0)

    z_e_flat, z_q_flat = pl.pallas_call(
        _tc_zq_body,
        grid=(G,),
        in_specs=[
            pl.BlockSpec((T, D_IN), lambda i: (i, 0)),
            pl.BlockSpec((D_IN, D_LAT), lambda i: (0, 0)),
            pl.BlockSpec((1, D_LAT), lambda i: (0, 0)),
            pl.BlockSpec((K, D_LAT), lambda i: (0, 0)),
            pl.BlockSpec((1, 1, T), lambda i: (i, 0, 0)),
        ],
        out_specs=[
            pl.BlockSpec((T, D_LAT), lambda i: (i, 0)),
            pl.BlockSpec((T, D_LAT), lambda i: (i, 0)),
        ],
        out_shape=[
            jax.ShapeDtypeStruct((N, D_LAT), jnp.float32),
            jax.ShapeDtypeStruct((N, D_LAT), jnp.float32),
        ],
    )(x2, W_enc, b_enc2, codebook, idx_all)

    x_rec_flat = jnp.concatenate([xr0, xr1], axis=0)

    return (z_e_flat.reshape(B, S, D_LAT),
            z_q_flat.reshape(B, S, D_LAT),
            x_rec_flat.reshape(B, S, D_IN),
            idx_all.reshape(B, S))
